# Initial kernel scaffold; baseline (speedup 1.0000x reference)
import jax
import jax.numpy as jnp
from jax.experimental import pallas as pl


def kernel(index, emb_weight):
    raise NotImplementedError("write your pallas kernel here")


# speedup vs baseline: 1.8761x; 1.8761x over previous
import functools

import jax
import jax.numpy as jnp
from jax import lax
from jax.experimental import pallas as pl
from jax.experimental.pallas import tpu as pltpu
from jax.experimental.pallas import tpu_sc as plsc

D = 64
SUB = 128
K = 4
CHUNK = K * SUB
NC = 2
NS = 16
NW = NC * NS
B = 16384 * 50
NBLK = B // SUB
KPW = NBLK // NW
NCH = KPW // K


@functools.partial(
    pl.kernel,
    out_type=jax.ShapeDtypeStruct((NBLK, SUB, D), jnp.float32),
    mesh=plsc.VectorSubcoreMesh(core_axis_name="c", subcore_axis_name="s"),
    compiler_params=pltpu.CompilerParams(use_tc_tiling_on_sc=False),
    scratch_types=[
        pltpu.VMEM((2, K, SUB), jnp.int32),
        pltpu.VMEM((2, K, SUB, D), jnp.float32),
        pltpu.SemaphoreType.DMA,
        pltpu.SemaphoreType.DMA,
        pltpu.SemaphoreType.DMA,
        pltpu.SemaphoreType.DMA,
    ],
)
def _emb_gather(idx_hbm, table_hbm, out_hbm, idx_v, rows_v, gsem0, gsem1,
                wsem0, wsem1):
    cid = lax.axis_index("c")
    sid = lax.axis_index("s")
    wid = sid * NC + cid
    base = wid * KPW

    gsems = (gsem0, gsem1)
    wsems = (wsem0, wsem1)

    def load_idx(c, buf):
        pltpu.sync_copy(idx_hbm.at[pl.ds(base + c * K, K)], idx_v.at[buf])

    def start_gathers(buf):
        for j in range(K):
            pltpu.async_copy(table_hbm.at[idx_v.at[buf, j]],
                             rows_v.at[buf, j], gsems[buf])

    def wait_gathers(buf):
        for j in range(K):
            pltpu.make_async_copy(table_hbm.at[idx_v.at[buf, j]],
                                  rows_v.at[buf, j], gsems[buf]).wait()

    def start_writeback(c, buf):
        pltpu.async_copy(rows_v.at[buf],
                         out_hbm.at[pl.ds(base + c * K, K)], wsems[buf])

    def wait_writeback(buf):
        pltpu.make_async_copy(rows_v.at[buf],
                              out_hbm.at[pl.ds(base, K)], wsems[buf]).wait()

    load_idx(0, 0)
    start_gathers(0)
    load_idx(1, 1)
    start_gathers(1)
    wait_gathers(0)
    start_writeback(0, 0)

    def pair(k, carry):
        for b, step in ((1, 0), (0, 1)):
            c = 1 + 2 * k + step
            nb = 1 - b
            load_idx(c + 1, nb)
            wait_writeback(nb)
            start_gathers(nb)
            wait_gathers(b)
            start_writeback(c, b)
        return carry

    lax.fori_loop(0, (NCH - 2) // 2, pair, 0)

    wait_gathers(1)
    start_writeback(NCH - 1, 1)
    wait_writeback(0)
    wait_writeback(1)


def kernel(index, emb_weight):
    idx2d = index.reshape(NBLK, SUB).astype(jnp.int32)
    out = _emb_gather(idx2d, emb_weight)
    return out.reshape(index.shape + (D,))
